# baseline (speedup 1.0000x reference)
import jax
import jax.numpy as jnp
from jax import lax
from jax.experimental import pallas as pl
from jax.experimental.pallas import tpu as pltpu
from jax.experimental.pallas import tpu_sc as plsc

NC = 2
NS = 16
NW = NC * NS
K = 80


def _transform1(h, W, b, N, D):
    BN = 1000

    def body(h_ref, w_ref, b_ref, hs_ref):
        acc = jnp.dot(h_ref[...], w_ref[...], preferred_element_type=jnp.float32)
        hs_ref[...] = acc + b_ref[...]

    return pl.pallas_call(
        body,
        grid=(N // BN,),
        in_specs=[
            pl.BlockSpec((BN, D), lambda i: (i, 0)),
            pl.BlockSpec((D, D), lambda i: (0, 0)),
            pl.BlockSpec((1, D), lambda i: (0, 0)),
        ],
        out_specs=pl.BlockSpec((BN, D), lambda i: (i, 0)),
        out_shape=jax.ShapeDtypeStruct((N, D), jnp.float32),
    )(h, W, b)


def _ctransform_wn(hs, p, inv, Wn, W2, bc, N, D):
    BN = 1000

    def body(hs_ref, p_ref, i_ref, wn_ref, w_ref, b_ref, hso_ref, z_ref):
        agg = (p_ref[0] + p_ref[1]) * i_ref[...]
        h = hs_ref[...] + jnp.dot(agg, wn_ref[...],
                                  preferred_element_type=jnp.float32)
        h = jnp.maximum(h, 0.0)
        acc = jnp.dot(h, w_ref[...], preferred_element_type=jnp.float32)
        acc = acc + b_ref[...]
        hso_ref[...] = acc[:, :D]
        z_ref[...] = acc[:, D:]

    return pl.pallas_call(
        body,
        grid=(N // BN,),
        in_specs=[
            pl.BlockSpec((BN, D), lambda i: (i, 0)),
            pl.BlockSpec((NC, BN, D), lambda i: (0, i, 0)),
            pl.BlockSpec((BN, D), lambda i: (i, 0)),
            pl.BlockSpec((D, D), lambda i: (0, 0)),
            pl.BlockSpec((D, 2 * D), lambda i: (0, 0)),
            pl.BlockSpec((1, 2 * D), lambda i: (0, 0)),
        ],
        out_specs=[
            pl.BlockSpec((BN, D), lambda i: (i, 0)),
            pl.BlockSpec((BN, D), lambda i: (i, 0)),
        ],
        out_shape=[jax.ShapeDtypeStruct((N, D), jnp.float32)] * 2,
    )(hs, p, inv, Wn, W2, bc)


def _inv_deg(degp, N, D):
    BN = 1000

    def body(d_ref, inv_ref):
        dsum = jnp.sum(d_ref[...], axis=1)
        inv = 1.0 / jnp.maximum(dsum, 1.0)
        inv_ref[...] = jnp.broadcast_to(inv[:, None], (BN, D))

    return pl.pallas_call(
        body,
        grid=(N // BN,),
        in_specs=[pl.BlockSpec((BN, NW), lambda i: (i, 0))],
        out_specs=pl.BlockSpec((BN, D), lambda i: (i, 0)),
        out_shape=jax.ShapeDtypeStruct((N, D), jnp.float32),
    )(degp)


def _combine(hs, p, inv, relu, N, D):
    BN = 1000

    def body(hs_ref, p_ref, i_ref, o_ref):
        o = hs_ref[...] + (p_ref[0] + p_ref[1]) * i_ref[...]
        o_ref[...] = jnp.maximum(o, 0.0) if relu else o

    return pl.pallas_call(
        body,
        grid=(N // BN,),
        in_specs=[
            pl.BlockSpec((BN, D), lambda i: (i, 0)),
            pl.BlockSpec((NC, BN, D), lambda i: (0, i, 0)),
            pl.BlockSpec((BN, D), lambda i: (i, 0)),
        ],
        out_specs=pl.BlockSpec((BN, D), lambda i: (i, 0)),
        out_shape=jax.ShapeDtypeStruct((N, D), jnp.float32),
    )(hs, p, inv)


def _ctransform(hs, p, inv, W2, bc, N, D):
    BN = 1000

    def body(hs_ref, p_ref, i_ref, w_ref, b_ref, hso_ref, z_ref):
        h = jnp.maximum(hs_ref[...] + (p_ref[0] + p_ref[1]) * i_ref[...], 0.0)
        acc = jnp.dot(h, w_ref[...], preferred_element_type=jnp.float32)
        acc = acc + b_ref[...]
        hso_ref[...] = acc[:, :D]
        z_ref[...] = acc[:, D:]

    return pl.pallas_call(
        body,
        grid=(N // BN,),
        in_specs=[
            pl.BlockSpec((BN, D), lambda i: (i, 0)),
            pl.BlockSpec((NC, BN, D), lambda i: (0, i, 0)),
            pl.BlockSpec((BN, D), lambda i: (i, 0)),
            pl.BlockSpec((D, 2 * D), lambda i: (0, 0)),
            pl.BlockSpec((1, 2 * D), lambda i: (0, 0)),
        ],
        out_specs=[
            pl.BlockSpec((BN, D), lambda i: (i, 0)),
            pl.BlockSpec((BN, D), lambda i: (i, 0)),
        ],
        out_shape=[jax.ShapeDtypeStruct((N, D), jnp.float32)] * 2,
    )(hs, p, inv, W2, bc)


def _make_deg(N, E):
    per_tile = E // NW
    n_chunks = per_tile // K
    mesh = plsc.VectorSubcoreMesh(core_axis_name="c", subcore_axis_name="s")

    def body(dst_hbm, zeros_hbm, deg_out, *, didx, deg_v):
        c = lax.axis_index("c")
        s = lax.axis_index("s")
        wid = c * NS + s
        pltpu.sync_copy(dst_hbm.at[pl.ds(wid * per_tile, per_tile)], didx)
        pltpu.sync_copy(zeros_hbm, deg_v)
        ones16 = jnp.ones((16,), jnp.float32)

        def chunk(i, _):
            dvec = didx[pl.ds(i * 16, 16)]
            plsc.addupdate_scatter(deg_v, [dvec], ones16)
            return 0

        lax.fori_loop(0, per_tile // 16, chunk, 0)
        pltpu.sync_copy(deg_v, deg_out.at[pl.ds(wid * N, N)])

    return pl.kernel(
        body,
        out_type=jax.ShapeDtypeStruct((NW * N,), jnp.float32),
        mesh=mesh,
        compiler_params=pltpu.CompilerParams(needs_layout_passes=False),
        scratch_types={
            "didx": pltpu.VMEM((per_tile,), jnp.int32),
            "deg_v": pltpu.VMEM((N,), jnp.float32),
        },
    )


def _make_aggregate(N, D, E):
    per_tile = E // NW
    n_chunks = per_tile // K
    share = (N // NS) // 8 * 8
    tail = N - share * NS
    chunks = [K] * (share // K)
    if share % K:
        chunks.append(share // K * K and share - share // K * K or share)
    chunks = [c for c in chunks if c > 0]

    mesh = plsc.VectorSubcoreMesh(core_axis_name="c", subcore_axis_name="s")

    def body(z_hbm, src_hbm, dst3_hbm, zeros_hbm, agg_out, *,
             sidx_all, didx_all, rows, sem, rows2, sem2, sem3, sem4, aggs):
        c = lax.axis_index("c")
        s = lax.axis_index("s")
        wid = c * NS + s

        pltpu.async_copy(src_hbm.at[pl.ds(wid * per_tile, per_tile)], sidx_all, sem)
        pltpu.async_copy(dst3_hbm.at[wid], didx_all, sem2)

        row0 = s * share
        pltpu.sync_copy(zeros_hbm, rows2)
        off = 0
        for n in chunks:
            pltpu.sync_copy(rows2.at[pl.ds(0, n)], aggs.at[pl.ds(row0 + off, n)])
            off += n

        @pl.when(s == 0)
        def _():
            pltpu.sync_copy(rows2.at[pl.ds(0, tail)],
                            aggs.at[pl.ds(NS * share, tail)])

        pltpu.make_async_copy(src_hbm.at[pl.ds(wid * per_tile, per_tile)],
                              sidx_all, sem).wait()
        pltpu.make_async_copy(dst3_hbm.at[wid], didx_all, sem2).wait()
        pltpu.async_copy(z_hbm.at[sidx_all.at[pl.ds(0, K)]], rows, sem)
        pltpu.async_copy(z_hbm.at[sidx_all.at[pl.ds(K, K)]], rows2, sem2)
        plsc.subcore_barrier()

        def pair(t, _):
            g0 = 2 * t
            pltpu.make_async_copy(z_hbm.at[sidx_all.at[pl.ds(g0 * K, K)]], rows, sem).wait()
            pltpu.async_copy(rows, aggs.at[didx_all.at[g0]], sem3, add=True)
            pltpu.make_async_copy(z_hbm.at[sidx_all.at[pl.ds((g0 + 1) * K, K)]], rows2, sem2).wait()
            pltpu.async_copy(rows2, aggs.at[didx_all.at[g0 + 1]], sem4, add=True)
            pltpu.make_async_copy(rows, aggs.at[didx_all.at[g0]], sem3).wait()

            @pl.when(g0 + 2 < n_chunks)
            def _():
                pltpu.async_copy(z_hbm.at[sidx_all.at[pl.ds((g0 + 2) * K, K)]], rows, sem)

            pltpu.make_async_copy(rows2, aggs.at[didx_all.at[g0 + 1]], sem4).wait()

            @pl.when(g0 + 3 < n_chunks)
            def _():
                pltpu.async_copy(z_hbm.at[sidx_all.at[pl.ds((g0 + 3) * K, K)]], rows2, sem2)

            return 0

        lax.fori_loop(0, n_chunks // 2, pair, 0)
        if n_chunks % 2 == 1:
            pltpu.make_async_copy(z_hbm.at[sidx_all.at[pl.ds((n_chunks - 1) * K, K)]], rows, sem).wait()
            pltpu.sync_copy(rows, aggs.at[didx_all.at[n_chunks - 1]], add=True)

        plsc.subcore_barrier()

        bufs = [(rows, sem), (rows2, sem2)]
        off = 0
        starts = []
        for i, n in enumerate(chunks):
            b, bs = bufs[i % 2]
            r0 = row0 + off
            pltpu.sync_copy(aggs.at[pl.ds(r0, n)], b.at[pl.ds(0, n)])
            pltpu.async_copy(b.at[pl.ds(0, n)], agg_out.at[pl.ds(c * N + r0, n)], bs)
            starts.append((b, bs, n, c * N + r0))
            if i >= 1:
                pb, pbs, pn, po = starts[i - 1]
                pltpu.make_async_copy(pb.at[pl.ds(0, pn)],
                                      agg_out.at[pl.ds(po, pn)], pbs).wait()
            off += n
        lb, lbs, ln, lo = starts[-1]
        pltpu.make_async_copy(lb.at[pl.ds(0, ln)],
                              agg_out.at[pl.ds(lo, ln)], lbs).wait()

        @pl.when(s == 0)
        def _():
            pltpu.sync_copy(aggs.at[pl.ds(NS * share, tail)], rows.at[pl.ds(0, tail)])
            pltpu.sync_copy(rows.at[pl.ds(0, tail)],
                            agg_out.at[pl.ds(c * N + NS * share, tail)])

    scratch = {
        "sidx_all": pltpu.VMEM((per_tile,), jnp.int32),
        "didx_all": pltpu.VMEM((n_chunks, K), jnp.int32),
        "rows": pltpu.VMEM((K, D), jnp.float32),
        "sem": pltpu.SemaphoreType.DMA,
        "rows2": pltpu.VMEM((K, D), jnp.float32),
        "sem2": pltpu.SemaphoreType.DMA,
        "sem3": pltpu.SemaphoreType.DMA,
        "sem4": pltpu.SemaphoreType.DMA,
        "aggs": pltpu.VMEM_SHARED((N, D), jnp.float32),
    }
    return pl.kernel(
        body,
        out_type=jax.ShapeDtypeStruct((NC * N, D), jnp.float32),
        mesh=mesh,
        scratch_types=scratch,
    )


def kernel(x, edge_index, Ws0, Wn0, b0, Ws1, Wn1, b1, Ws2, Wn2, b2):
    N, D = x.shape
    E = edge_index.shape[1]
    src = edge_index[0]
    dst = edge_index[1]

    agg = _make_aggregate(N, D, E)
    deg_k = _make_deg(N, E)
    zeros_d = jnp.zeros((K, D), jnp.float32)
    zeros_n = jnp.zeros((N,), jnp.float32)

    W21 = jnp.concatenate([Ws1, Wn1], axis=1)
    W22 = jnp.concatenate([Ws2, Wn2], axis=1)
    zcol = jnp.zeros((D,), jnp.float32)
    bc1 = jnp.concatenate([b1, zcol])[None, :]
    bc2 = jnp.concatenate([b2, zcol])[None, :]

    n_chunks = (E // NW) // K
    dst3 = dst.reshape(NW, n_chunks, K)

    degp = deg_k(dst, zeros_n).reshape(NW, N).T
    p0 = agg(x, src, dst3, zeros_d)
    hs0 = _transform1(x, Ws0, b0[None, :], N, D)
    inv = _inv_deg(degp, N, D)
    hs1, z1 = _ctransform_wn(hs0, p0.reshape(NC, N, D), inv, Wn0, W21, bc1,
                             N, D)
    p1 = agg(z1, src, dst3, zeros_d)
    hs2, z2 = _ctransform(hs1, p1.reshape(NC, N, D), inv, W22, bc2, N, D)
    p2 = agg(z2, src, dst3, zeros_d)
    return _combine(hs2, p2.reshape(NC, N, D), inv, False, N, D)


# speedup vs baseline: 1.2480x; 1.2480x over previous
import jax
import jax.numpy as jnp
from jax import lax
from jax.experimental import pallas as pl
from jax.experimental.pallas import tpu as pltpu
from jax.experimental.pallas import tpu_sc as plsc

NC = 2
NS = 16
NW = NC * NS
K = 80


def _transform1(h, W, b, N, D):
    BN = 1000

    def body(h_ref, w_ref, b_ref, hs_ref):
        acc = jnp.dot(h_ref[...], w_ref[...], preferred_element_type=jnp.float32)
        hs_ref[...] = acc + b_ref[...]

    return pl.pallas_call(
        body,
        grid=(N // BN,),
        in_specs=[
            pl.BlockSpec((BN, D), lambda i: (i, 0)),
            pl.BlockSpec((D, D), lambda i: (0, 0)),
            pl.BlockSpec((1, D), lambda i: (0, 0)),
        ],
        out_specs=pl.BlockSpec((BN, D), lambda i: (i, 0)),
        out_shape=jax.ShapeDtypeStruct((N, D), jnp.float32),
    )(h, W, b)


def _ctransform_wn(hs, p, inv, Wn, W2, bc, N, D):
    BN = 1000

    def body(hs_ref, p_ref, i_ref, wn_ref, w_ref, b_ref, hso_ref, z_ref):
        agg = (p_ref[0] + p_ref[1]) * i_ref[...]
        h = hs_ref[...] + jnp.dot(agg, wn_ref[...],
                                  preferred_element_type=jnp.float32)
        h = jnp.maximum(h, 0.0)
        acc = jnp.dot(h, w_ref[...], preferred_element_type=jnp.float32)
        acc = acc + b_ref[...]
        hso_ref[...] = acc[:, :D]
        z_ref[...] = acc[:, D:]

    return pl.pallas_call(
        body,
        grid=(N // BN,),
        in_specs=[
            pl.BlockSpec((BN, D), lambda i: (i, 0)),
            pl.BlockSpec((NC, BN, D), lambda i: (0, i, 0)),
            pl.BlockSpec((BN, D), lambda i: (i, 0)),
            pl.BlockSpec((D, D), lambda i: (0, 0)),
            pl.BlockSpec((D, 2 * D), lambda i: (0, 0)),
            pl.BlockSpec((1, 2 * D), lambda i: (0, 0)),
        ],
        out_specs=[
            pl.BlockSpec((BN, D), lambda i: (i, 0)),
            pl.BlockSpec((BN, D), lambda i: (i, 0)),
        ],
        out_shape=[jax.ShapeDtypeStruct((N, D), jnp.float32)] * 2,
    )(hs, p, inv, Wn, W2, bc)


def _inv_deg(degp, N, D):
    BN = 1000

    def body(d_ref, inv_ref):
        dsum = jnp.sum(d_ref[...], axis=1)
        inv = 1.0 / jnp.maximum(dsum, 1.0)
        inv_ref[...] = jnp.broadcast_to(inv[:, None], (BN, D))

    return pl.pallas_call(
        body,
        grid=(N // BN,),
        in_specs=[pl.BlockSpec((BN, NW), lambda i: (i, 0))],
        out_specs=pl.BlockSpec((BN, D), lambda i: (i, 0)),
        out_shape=jax.ShapeDtypeStruct((N, D), jnp.float32),
    )(degp)


def _combine(hs, p, inv, relu, N, D):
    BN = 1000

    def body(hs_ref, p_ref, i_ref, o_ref):
        o = hs_ref[...] + (p_ref[0] + p_ref[1]) * i_ref[...]
        o_ref[...] = jnp.maximum(o, 0.0) if relu else o

    return pl.pallas_call(
        body,
        grid=(N // BN,),
        in_specs=[
            pl.BlockSpec((BN, D), lambda i: (i, 0)),
            pl.BlockSpec((NC, BN, D), lambda i: (0, i, 0)),
            pl.BlockSpec((BN, D), lambda i: (i, 0)),
        ],
        out_specs=pl.BlockSpec((BN, D), lambda i: (i, 0)),
        out_shape=jax.ShapeDtypeStruct((N, D), jnp.float32),
    )(hs, p, inv)


def _ctransform(hs, p, inv, W2, bc, N, D):
    BN = 1000

    def body(hs_ref, p_ref, i_ref, w_ref, b_ref, hso_ref, z_ref):
        h = jnp.maximum(hs_ref[...] + (p_ref[0] + p_ref[1]) * i_ref[...], 0.0)
        acc = jnp.dot(h, w_ref[...], preferred_element_type=jnp.float32)
        acc = acc + b_ref[...]
        hso_ref[...] = acc[:, :D]
        z_ref[...] = acc[:, D:]

    return pl.pallas_call(
        body,
        grid=(N // BN,),
        in_specs=[
            pl.BlockSpec((BN, D), lambda i: (i, 0)),
            pl.BlockSpec((NC, BN, D), lambda i: (0, i, 0)),
            pl.BlockSpec((BN, D), lambda i: (i, 0)),
            pl.BlockSpec((D, 2 * D), lambda i: (0, 0)),
            pl.BlockSpec((1, 2 * D), lambda i: (0, 0)),
        ],
        out_specs=[
            pl.BlockSpec((BN, D), lambda i: (i, 0)),
            pl.BlockSpec((BN, D), lambda i: (i, 0)),
        ],
        out_shape=[jax.ShapeDtypeStruct((N, D), jnp.float32)] * 2,
    )(hs, p, inv, W2, bc)


def _make_deg(N, E):
    per_tile = E // NW
    n_chunks = per_tile // K
    mesh = plsc.VectorSubcoreMesh(core_axis_name="c", subcore_axis_name="s")

    def body(dst_hbm, zeros_hbm, deg_out, *, didx, deg_v):
        c = lax.axis_index("c")
        s = lax.axis_index("s")
        wid = c * NS + s
        pltpu.sync_copy(dst_hbm.at[pl.ds(wid * per_tile, per_tile)], didx)
        pltpu.sync_copy(zeros_hbm, deg_v)
        ones16 = jnp.ones((16,), jnp.float32)

        def chunk(i, _):
            dvec = didx[pl.ds(i * 16, 16)]
            plsc.addupdate_scatter(deg_v, [dvec], ones16)
            return 0

        lax.fori_loop(0, per_tile // 16, chunk, 0)
        pltpu.sync_copy(deg_v, deg_out.at[pl.ds(wid * N, N)])

    return pl.kernel(
        body,
        out_type=jax.ShapeDtypeStruct((NW * N,), jnp.float32),
        mesh=mesh,
        compiler_params=pltpu.CompilerParams(needs_layout_passes=False),
        scratch_types={
            "didx": pltpu.VMEM((per_tile,), jnp.int32),
            "deg_v": pltpu.VMEM((N,), jnp.float32),
        },
    )


def _make_aggregate(N, D, E):
    per_tile = E // NW
    n_chunks = per_tile // K
    share = (N // NS) // 8 * 8
    tail = N - share * NS
    chunks = [K] * (share // K)
    if share % K:
        chunks.append(share // K * K and share - share // K * K or share)
    chunks = [c for c in chunks if c > 0]

    mesh = plsc.VectorSubcoreMesh(core_axis_name="c", subcore_axis_name="s")

    def body(z_hbm, src_hbm, dst3_hbm, zeros_hbm, agg_out, *,
             sidx_all, didx_all, rows, sem, rows2, sem2, aggs):
        c = lax.axis_index("c")
        s = lax.axis_index("s")
        wid = c * NS + s

        pltpu.async_copy(src_hbm.at[pl.ds(wid * per_tile, per_tile)], sidx_all, sem)
        pltpu.async_copy(dst3_hbm.at[wid], didx_all, sem2)

        row0 = s * share
        pltpu.sync_copy(zeros_hbm, rows2)
        off = 0
        for n in chunks:
            pltpu.sync_copy(rows2.at[pl.ds(0, n)], aggs.at[pl.ds(row0 + off, n)])
            off += n

        @pl.when(s == 0)
        def _():
            pltpu.sync_copy(rows2.at[pl.ds(0, tail)],
                            aggs.at[pl.ds(NS * share, tail)])

        pltpu.make_async_copy(src_hbm.at[pl.ds(wid * per_tile, per_tile)],
                              sidx_all, sem).wait()
        pltpu.make_async_copy(dst3_hbm.at[wid], didx_all, sem2).wait()
        pltpu.async_copy(z_hbm.at[sidx_all.at[pl.ds(0, K)]], rows, sem)
        plsc.subcore_barrier()

        def pair(t, _):
            g0 = 2 * t
            pltpu.async_copy(z_hbm.at[sidx_all.at[pl.ds((g0 + 1) * K, K)]], rows2, sem2)
            pltpu.make_async_copy(z_hbm.at[sidx_all.at[pl.ds(g0 * K, K)]], rows, sem).wait()
            pltpu.sync_copy(rows, aggs.at[didx_all.at[g0]], add=True)

            @pl.when(g0 + 2 < n_chunks)
            def _():
                pltpu.async_copy(z_hbm.at[sidx_all.at[pl.ds((g0 + 2) * K, K)]], rows, sem)

            pltpu.make_async_copy(z_hbm.at[sidx_all.at[pl.ds((g0 + 1) * K, K)]], rows2, sem2).wait()
            pltpu.sync_copy(rows2, aggs.at[didx_all.at[g0 + 1]], add=True)
            return 0

        lax.fori_loop(0, n_chunks // 2, pair, 0)
        if n_chunks % 2 == 1:
            pltpu.make_async_copy(z_hbm.at[sidx_all.at[pl.ds((n_chunks - 1) * K, K)]], rows, sem).wait()
            pltpu.sync_copy(rows, aggs.at[didx_all.at[n_chunks - 1]], add=True)

        plsc.subcore_barrier()

        bufs = [(rows, sem), (rows2, sem2)]
        off = 0
        starts = []
        for i, n in enumerate(chunks):
            b, bs = bufs[i % 2]
            r0 = row0 + off
            pltpu.sync_copy(aggs.at[pl.ds(r0, n)], b.at[pl.ds(0, n)])
            pltpu.async_copy(b.at[pl.ds(0, n)], agg_out.at[pl.ds(c * N + r0, n)], bs)
            starts.append((b, bs, n, c * N + r0))
            if i >= 1:
                pb, pbs, pn, po = starts[i - 1]
                pltpu.make_async_copy(pb.at[pl.ds(0, pn)],
                                      agg_out.at[pl.ds(po, pn)], pbs).wait()
            off += n
        lb, lbs, ln, lo = starts[-1]
        pltpu.make_async_copy(lb.at[pl.ds(0, ln)],
                              agg_out.at[pl.ds(lo, ln)], lbs).wait()

        @pl.when(s == 0)
        def _():
            pltpu.sync_copy(aggs.at[pl.ds(NS * share, tail)], rows.at[pl.ds(0, tail)])
            pltpu.sync_copy(rows.at[pl.ds(0, tail)],
                            agg_out.at[pl.ds(c * N + NS * share, tail)])

    scratch = {
        "sidx_all": pltpu.VMEM((per_tile,), jnp.int32),
        "didx_all": pltpu.VMEM((n_chunks, K), jnp.int32),
        "rows": pltpu.VMEM((K, D), jnp.float32),
        "sem": pltpu.SemaphoreType.DMA,
        "rows2": pltpu.VMEM((K, D), jnp.float32),
        "sem2": pltpu.SemaphoreType.DMA,
        "aggs": pltpu.VMEM_SHARED((N, D), jnp.float32),
    }
    return pl.kernel(
        body,
        out_type=jax.ShapeDtypeStruct((NC * N, D), jnp.float32),
        mesh=mesh,
        scratch_types=scratch,
    )


def kernel(x, edge_index, Ws0, Wn0, b0, Ws1, Wn1, b1, Ws2, Wn2, b2):
    N, D = x.shape
    E = edge_index.shape[1]
    src = edge_index[0]
    dst = edge_index[1]

    agg = _make_aggregate(N, D, E)
    deg_k = _make_deg(N, E)
    zeros_d = jnp.zeros((K, D), jnp.float32)
    zeros_n = jnp.zeros((N,), jnp.float32)

    W21 = jnp.concatenate([Ws1, Wn1], axis=1)
    W22 = jnp.concatenate([Ws2, Wn2], axis=1)
    zcol = jnp.zeros((D,), jnp.float32)
    bc1 = jnp.concatenate([b1, zcol])[None, :]
    bc2 = jnp.concatenate([b2, zcol])[None, :]

    n_chunks = (E // NW) // K
    dst3 = dst.reshape(NW, n_chunks, K)

    degp = deg_k(dst, zeros_n).reshape(NW, N).T
    p0 = agg(x, src, dst3, zeros_d)
    hs0 = _transform1(x, Ws0, b0[None, :], N, D)
    inv = _inv_deg(degp, N, D)
    hs1, z1 = _ctransform_wn(hs0, p0.reshape(NC, N, D), inv, Wn0, W21, bc1,
                             N, D)
    p1 = agg(z1, src, dst3, zeros_d)
    hs2, z2 = _ctransform(hs1, p1.reshape(NC, N, D), inv, W22, bc2, N, D)
    p2 = agg(z2, src, dst3, zeros_d)
    return _combine(hs2, p2.reshape(NC, N, D), inv, False, N, D)
